# baseline (speedup 1.0000x reference)
import functools

import jax
import jax.numpy as jnp
from jax import lax
from jax.experimental import pallas as pl
from jax.experimental.pallas import tpu as pltpu
from jax.experimental.pallas import tpu_sc as plsc

_L = 128
_K = 8


@functools.partial(jax.jit, static_argnums=(2, 3))
def _sc_embedding_gather(idx2d, table, n_workers, rows_per_w):
    d = table.shape[1]
    steps = rows_per_w // _K
    mesh = plsc.VectorSubcoreMesh(core_axis_name="c", subcore_axis_name="s")
    n_cores = mesh.num_cores

    @functools.partial(
        pl.kernel,
        mesh=mesh,
        out_type=jax.ShapeDtypeStruct((idx2d.shape[0] * _L, d), jnp.float32),
        scratch_types=[
            pltpu.VMEM((_K, _L), jnp.int32),
            pltpu.VMEM((_K * _L, d), jnp.float32),
            pltpu.SemaphoreType.DMA,
        ],
        compiler_params=pltpu.CompilerParams(use_tc_tiling_on_sc=False),
    )
    def k(idx_hbm, table_hbm, out_hbm, idx_v, rows_v, sem):
        wid = lax.axis_index("s") * n_cores + lax.axis_index("c")
        row0 = wid * rows_per_w

        def body(g, carry):
            r = row0 + g * _K
            pltpu.sync_copy(idx_hbm.at[pl.ds(r, _K)], idx_v)
            copies = [
                pltpu.async_copy(
                    table_hbm.at[idx_v.at[j]],
                    rows_v.at[pl.ds(j * _L, _L)],
                    sem,
                )
                for j in range(_K)
            ]
            for cp in copies:
                cp.wait()
            pltpu.sync_copy(rows_v, out_hbm.at[pl.ds(r * _L, _K * _L)])
            return carry

        lax.fori_loop(0, steps, body, 0)

    return k(idx2d, table)


def kernel(x, table):
    b, h = x.shape
    d = table.shape[1]
    total = b * h
    n_workers = 32
    rows = total // _L
    rows_per_w = rows // n_workers
    idx2d = x.reshape(rows, _L).astype(jnp.int32)
    flat = _sc_embedding_gather(idx2d, table, n_workers, rows_per_w)
    return flat.reshape(b, h, d)
